# baseline (speedup 1.0000x reference)
import jax
import jax.numpy as jnp
from jax.experimental import pallas as pl


def _bcast_body(t_ref, o_ref):
    o_ref[...] = jnp.broadcast_to(t_ref[...][None], o_ref.shape)


def kernel(x, table):
    B, T, N, _ = x.shape
    D = table.shape[1]
    R = B * T
    rows = N * D // 128
    t2 = table.reshape(rows, 128)
    C = 4
    out = pl.pallas_call(
        _bcast_body,
        grid=(R // C,),
        in_specs=[pl.BlockSpec((rows, 128), lambda i: (0, 0))],
        out_specs=pl.BlockSpec((C, rows, 128), lambda i: (i, 0, 0)),
        out_shape=jax.ShapeDtypeStruct((R, rows, 128), jnp.float32),
    )(t2)
    return out.reshape(B, T, N, D)
